# baseline (speedup 1.0000x reference)
import functools

import jax
import jax.numpy as jnp
from jax import lax
from jax.experimental import pallas as pl
from jax.experimental.pallas import tpu as pltpu
from jax.experimental.pallas import tpu_sc as plsc

_VOCAB = 1000000
_EMBED = 64
_BATCH = 16384
_C = 5

_NC = 2
_NS = 16
_NW = _NC * _NS
_BPW = _BATCH // _NW
_CB = 128
_NCHUNK = _BPW // _CB
_IW = 128
_W = 2 * _EMBED


def _dots_kernel(tt_hbm, ct_hbm, tgt_hbm, ctx_hbm, out_hbm,
                 idx_t, idx_c, idx_ts, idx_cs, rows_t, rows_c, out_v, sem):
    wid = lax.axis_index("s") * _NC + lax.axis_index("c")
    base = wid * _BPW

    def chunk_body(ch, _):
        b0 = base + ch * _CB
        pltpu.sync_copy(tgt_hbm.at[pl.ds(b0, _CB)], idx_t)
        pltpu.sync_copy(ctx_hbm.at[pl.ds(b0 * _C, _CB * _C)], idx_c)

        for j in range(_CB // 16):
            idx_ts[pl.ds(j * 16, 16)] = lax.shift_right_logical(
                idx_t[pl.ds(j * 16, 16)], 1)
        for j in range(_CB * _C // 16):
            idx_cs[pl.ds(j * 16, 16)] = lax.shift_right_logical(
                idx_c[pl.ds(j * 16, 16)], 1)

        copies = []
        for j in range(_CB // _IW):
            copies.append(pltpu.async_copy(
                tt_hbm.at[idx_ts.at[pl.ds(j * _IW, _IW)]],
                rows_t.at[pl.ds(j * _IW, _IW)], sem))
        for j in range(_CB * _C // _IW):
            copies.append(pltpu.async_copy(
                ct_hbm.at[idx_cs.at[pl.ds(j * _IW, _IW)]],
                rows_c.at[pl.ds(j * _IW, _IW)], sem))
        for cp in copies:
            cp.wait()

        def bg_body(bg, _):
            bvec = lax.iota(jnp.int32, 16) + bg * 16
            tpar = lax.shift_left(
                jnp.bitwise_and(idx_t[pl.ds(bg * 16, 16)], 1), 6)
            crow = [bvec * _C + c for c in range(_C)]
            cpar = [lax.shift_left(
                jnp.bitwise_and(
                    plsc.load_gather(idx_c, [crow[c]]), 1), 6)
                for c in range(_C)]
            acc = [jnp.zeros((16,), jnp.float32) for _ in range(_C)]
            for e in range(_EMBED):
                wv = plsc.load_gather(rows_t, [bvec, tpar + e])
                for c in range(_C):
                    cv = plsc.load_gather(rows_c, [crow[c], cpar[c] + e])
                    acc[c] = acc[c] + wv * cv
            for c in range(_C):
                plsc.store_scatter(out_v, [crow[c]], acc[c])
            return _

        lax.fori_loop(0, _CB // 16, bg_body, None)

        pltpu.sync_copy(out_v, out_hbm.at[pl.ds(b0 * _C, _CB * _C)])
        return _

    lax.fori_loop(0, _NCHUNK, chunk_body, None)


@jax.jit
def _run(target, context, target_table, context_table):
    mesh = plsc.VectorSubcoreMesh(core_axis_name="c", subcore_axis_name="s",
                                  num_cores=_NC, num_subcores=_NS)
    k = functools.partial(
        pl.kernel,
        out_type=jax.ShapeDtypeStruct((_BATCH * _C,), jnp.float32),
        mesh=mesh,
        compiler_params=pltpu.CompilerParams(needs_layout_passes=False),
        scratch_types=[
            pltpu.VMEM((_CB,), jnp.int32),
            pltpu.VMEM((_CB * _C,), jnp.int32),
            pltpu.VMEM((_CB,), jnp.int32),
            pltpu.VMEM((_CB * _C,), jnp.int32),
            pltpu.VMEM((_CB, _W), jnp.float32),
            pltpu.VMEM((_CB * _C, _W), jnp.float32),
            pltpu.VMEM((_CB * _C,), jnp.float32),
            pltpu.SemaphoreType.DMA,
        ],
    )(_dots_kernel)
    ct2 = jnp.concatenate([context_table[0::2], context_table[1::2]], axis=1)
    flat = k(target_table.reshape(_VOCAB // 2, _W),
             ct2,
             target, context.reshape(-1))
    return flat.reshape(_BATCH, _C)


def kernel(target, context, target_table, context_table):
    if target.ndim == 2:
        target = jnp.squeeze(target, axis=1)
    return _run(target.astype(jnp.int32), context.astype(jnp.int32),
                target_table, context_table)


# speedup vs baseline: 12.9430x; 12.9430x over previous
import functools

import jax
import jax.numpy as jnp
from jax import lax
from jax.experimental import pallas as pl
from jax.experimental.pallas import tpu as pltpu
from jax.experimental.pallas import tpu_sc as plsc

_VOCAB = 1000000
_EMBED = 64
_BATCH = 16384
_C = 5

_NC = 2
_NS = 16
_NW = _NC * _NS
_BPW = _BATCH // _NW
_CB = 16
_NCHUNK = _BPW // _CB
_BLK = 8


def _dots_kernel(tt_hbm, ct_hbm, tgt_hbm, ctx_hbm, out_hbm,
                 idx_t, idx_c, rows_t, rows_c, out_v, sem):
    wid = lax.axis_index("s") * _NC + lax.axis_index("c")
    base = wid * _BPW

    def chunk_body(ch, _):
        b0 = base + ch * _CB
        pltpu.sync_copy(tgt_hbm.at[pl.ds(b0, _CB)], idx_t)
        pltpu.sync_copy(ctx_hbm.at[pl.ds(b0 * _C, _CB * _C)], idx_c)

        tvec = idx_t[pl.ds(0, 16)]
        tblk = lax.shift_right_logical(tvec, 3)
        copies = []
        for j in range(_CB):
            copies.append(pltpu.async_copy(
                tt_hbm.at[tblk[j]], rows_t.at[j], sem))
        cblks = []
        for g in range(_CB * _C // 16):
            cvec = idx_c[pl.ds(g * 16, 16)]
            cblks.append(lax.shift_right_logical(cvec, 3))
        for g in range(_CB * _C // 16):
            for j in range(16):
                copies.append(pltpu.async_copy(
                    ct_hbm.at[cblks[g][j]], rows_c.at[g * 16 + j], sem))
        for cp in copies:
            cp.wait()

        bvec = lax.iota(jnp.int32, 16)
        tsub = jnp.bitwise_and(tvec, 7)
        crow = [bvec * _C + c for c in range(_C)]
        csub = [jnp.bitwise_and(
            plsc.load_gather(idx_c, [crow[c]]), 7) for c in range(_C)]
        acc = [jnp.zeros((16,), jnp.float32) for _ in range(_C)]
        for e in range(_EMBED):
            ecol = jnp.full((16,), e, jnp.int32)
            wv = plsc.load_gather(rows_t, [bvec, tsub, ecol])
            for c in range(_C):
                cv = plsc.load_gather(rows_c, [crow[c], csub[c], ecol])
                acc[c] = acc[c] + wv * cv
        for c in range(_C):
            plsc.store_scatter(out_v, [crow[c]], acc[c])

        pltpu.sync_copy(out_v, out_hbm.at[pl.ds(b0 * _C, _CB * _C)])
        return _

    lax.fori_loop(0, _NCHUNK, chunk_body, None)


@jax.jit
def _run(target, context, target_table, context_table):
    mesh = plsc.VectorSubcoreMesh(core_axis_name="c", subcore_axis_name="s",
                                  num_cores=_NC, num_subcores=_NS)
    k = functools.partial(
        pl.kernel,
        out_type=jax.ShapeDtypeStruct((_BATCH * _C,), jnp.float32),
        mesh=mesh,
        compiler_params=pltpu.CompilerParams(needs_layout_passes=False),
        scratch_types=[
            pltpu.VMEM((_CB,), jnp.int32),
            pltpu.VMEM((_CB * _C,), jnp.int32),
            pltpu.VMEM((_CB, _BLK, _EMBED), jnp.float32),
            pltpu.VMEM((_CB * _C, _BLK, _EMBED), jnp.float32),
            pltpu.VMEM((_CB * _C,), jnp.float32),
            pltpu.SemaphoreType.DMA,
        ],
    )(_dots_kernel)
    flat = k(target_table.reshape(_VOCAB // _BLK, _BLK, _EMBED),
             context_table.reshape(_VOCAB // _BLK, _BLK, _EMBED),
             target, context.reshape(-1))
    return flat.reshape(_BATCH, _C)


def kernel(target, context, target_table, context_table):
    if target.ndim == 2:
        target = jnp.squeeze(target, axis=1)
    return _run(target.astype(jnp.int32), context.astype(jnp.int32),
                target_table, context_table)
